# baseline (speedup 1.0000x reference)
import functools

import jax
import jax.numpy as jnp
from jax import lax
from jax.experimental import pallas as pl
from jax.experimental.pallas import tpu as pltpu
from jax.experimental.pallas import tpu_sc as plsc

NUM_EMBEDDINGS = 1000000
FEATURES = 64
BATCH = 16384
FIELDS = 26

NC = 2
NS = 16
NW = NC * NS
LANES = 16

CHUNK = 128
NBLK_TOTAL = BATCH * FIELDS // CHUNK
NBLK = NBLK_TOTAL // NW
TC_TILES = FEATURES // 8
BCOLS = BATCH // CHUNK

_mesh = plsc.VectorSubcoreMesh(core_axis_name="c", subcore_axis_name="s")


@functools.partial(
    pl.kernel,
    mesh=_mesh,
    out_type=jax.ShapeDtypeStruct(
        (FIELDS, TC_TILES, BCOLS, 8, CHUNK), jnp.float32
    ),
    scratch_types=[
        pltpu.VMEM((3, CHUNK), jnp.int32),
        pltpu.VMEM((2, CHUNK), jnp.int32),
        pltpu.VMEM((2, CHUNK, 128), jnp.float32),
        pltpu.VMEM((2, FEATURES, CHUNK), jnp.float32),
        pltpu.SemaphoreType.DMA,
        pltpu.SemaphoreType.DMA,
        pltpu.SemaphoreType.DMA,
    ],
    compiler_params=pltpu.CompilerParams(
        use_tc_tiling_on_sc=True, needs_layout_passes=False
    ),
)
def _embed_gather(idx_hbm, table_hbm, out_hbm, idx_v, idx2_v, rows_v, t_v,
                  isem, gsem, ssem):
    wid = lax.axis_index("s") * NC + lax.axis_index("c")
    blk0 = wid * NBLK
    lane = lax.iota(jnp.int32, LANES)

    def blk_ft(k):
        blk = blk0 + k
        return blk // BCOLS, lax.rem(blk, BCOLS)

    def idx_src(k):
        f, tb = blk_ft(k)
        return idx_hbm.at[f, pl.ds(tb * CHUNK, CHUNK)]

    def fill_idx2(k):
        src = idx_v.at[lax.rem(k, 3)]
        dst = idx2_v.at[lax.rem(k, 2)]
        for j in range(CHUNK // LANES):
            v = src[pl.ds(j * LANES, LANES)]
            dst[pl.ds(j * LANES, LANES)] = lax.shift_right_logical(v, 1)

    def start_gather(k):
        pltpu.async_copy(
            table_hbm.at[idx2_v.at[lax.rem(k, 2)]],
            rows_v.at[lax.rem(k, 2)],
            gsem,
        )

    pltpu.sync_copy(idx_src(0), idx_v.at[0])
    fill_idx2(0)
    start_gather(0)
    pltpu.async_copy(idx_src(1), idx_v.at[1], isem)

    def body(k, carry):
        buf = lax.rem(k, 2)

        @pl.when(k + 1 < NBLK)
        def _():
            pltpu.make_async_copy(
                idx_src(k + 1), idx_v.at[lax.rem(k + 1, 3)], isem
            ).wait()
            fill_idx2(k + 1)
            start_gather(k + 1)

        @pl.when(k + 2 < NBLK)
        def _():
            pltpu.async_copy(idx_src(k + 2), idx_v.at[lax.rem(k + 2, 3)], isem)

        pltpu.make_async_copy(
            table_hbm.at[idx2_v.at[buf]], rows_v.at[buf], gsem
        ).wait()

        f, tb = blk_ft(k)

        @pl.when(k >= 2)
        def _():
            for tc in range(TC_TILES):
                pltpu.make_async_copy(
                    t_v.at[buf, pl.ds(tc * 8, 8)], out_hbm.at[f, tc, tb], ssem
                ).wait()

        rbuf = rows_v.at[buf]
        tbuf = t_v.at[buf]
        src = idx_v.at[lax.rem(k, 3)]
        for grp in range(CHUNK // LANES):
            pv = src[pl.ds(grp * LANES, LANES)]
            cvec0 = lax.mul(lax.rem(pv, 2), jnp.int32(FEATURES))
            rv = rbuf.at[pl.ds(grp * LANES, LANES)]
            for c in range(FEATURES):
                vals = plsc.load_gather(rv, [lane, cvec0 + c])
                tbuf[c, pl.ds(grp * LANES, LANES)] = vals

        for tc in range(TC_TILES):
            pltpu.async_copy(
                tbuf.at[pl.ds(tc * 8, 8)], out_hbm.at[f, tc, tb], ssem
            )
        return carry

    lax.fori_loop(0, NBLK, body, 0)

    for _ in range(2 * TC_TILES):
        pltpu.make_async_copy(
            t_v.at[0, pl.ds(0, 8)], out_hbm.at[0, 0, 0], ssem
        ).wait()


def kernel(inputs, embedding):
    idx_t = inputs.astype(jnp.int32).T
    table2 = jnp.asarray(embedding, jnp.float32).reshape(
        NUM_EMBEDDINGS // 2, 2 * FEATURES
    )
    out5 = _embed_gather(idx_t, table2)
    return out5.transpose(2, 4, 0, 1, 3).reshape(BATCH, FIELDS, FEATURES)


# speedup vs baseline: 1.3776x; 1.3776x over previous
import functools

import jax
import jax.numpy as jnp
from jax import lax
from jax.experimental import pallas as pl
from jax.experimental.pallas import tpu as pltpu
from jax.experimental.pallas import tpu_sc as plsc

NUM_EMBEDDINGS = 1000000
FEATURES = 64
BATCH = 16384
FIELDS = 26

NC = 2
NS = 16
NW = NC * NS

B_TOTAL = BATCH * FIELDS
BPW = B_TOTAL // NW
CHUNK = 128
NCHUNK = BPW // CHUNK
K = 4
NGROUP = NCHUNK // K
NBUF = 3 * K
BCOLS = BATCH // CHUNK

_mesh = plsc.VectorSubcoreMesh(core_axis_name="c", subcore_axis_name="s")


@functools.partial(
    pl.kernel,
    mesh=_mesh,
    out_type=jax.ShapeDtypeStruct((B_TOTAL,), jnp.int32),
    scratch_types=[
        pltpu.VMEM((BPW,), jnp.int32),
        pltpu.SemaphoreType.DMA,
    ],
    compiler_params=pltpu.CompilerParams(use_tc_tiling_on_sc=True),
)
def _detile_idx(idx_hbm, out_hbm, buf_v, sem):
    wid = lax.axis_index("s") * NC + lax.axis_index("c")
    blk0 = wid * NCHUNK

    def src(k):
        blk = blk0 + k
        return idx_hbm.at[blk // BCOLS, pl.ds(lax.rem(blk, BCOLS) * CHUNK, CHUNK)]

    def body(g, carry):
        for j in range(8):
            k = g * 8 + j
            pltpu.async_copy(src(k), buf_v.at[pl.ds(k * CHUNK, CHUNK)], sem)
        for j in range(8):
            k = g * 8 + j
            pltpu.make_async_copy(
                src(k), buf_v.at[pl.ds(k * CHUNK, CHUNK)], sem
            ).wait()
        return carry

    lax.fori_loop(0, NCHUNK // 8, body, 0)
    pltpu.sync_copy(buf_v, out_hbm.at[pl.ds(wid * BPW, BPW)])


@functools.partial(
    pl.kernel,
    mesh=_mesh,
    out_type=jax.ShapeDtypeStruct((B_TOTAL, FEATURES), jnp.float32),
    scratch_types=[
        pltpu.VMEM((BPW,), jnp.int32),
        pltpu.VMEM((NBUF, CHUNK, FEATURES), jnp.float32),
        pltpu.SemaphoreType.DMA,
        pltpu.SemaphoreType.DMA,
    ],
    compiler_params=pltpu.CompilerParams(use_tc_tiling_on_sc=False),
)
def _embed_gather(idx_hbm, table_hbm, out_hbm, idx_v, rows_v, gsem, ssem):
    wid = lax.axis_index("s") * NC + lax.axis_index("c")
    base = wid * BPW

    pltpu.sync_copy(idx_hbm.at[pl.ds(base, BPW)], idx_v)

    def idx_ref(c):
        return idx_v.at[pl.ds(c * CHUNK, CHUNK)]

    for j in range(K):
        pltpu.async_copy(table_hbm.at[idx_ref(j)], rows_v.at[j], gsem)

    def body(i, carry):
        @pl.when(i >= 2)
        def _():
            for j in range(K):
                pltpu.make_async_copy(
                    rows_v.at[j], out_hbm.at[pl.ds(base, CHUNK)], ssem
                ).wait()

        @pl.when(i + 1 < NGROUP)
        def _():
            boff = lax.rem(i + 1, 3) * K
            for j in range(K):
                pltpu.async_copy(
                    table_hbm.at[idx_ref((i + 1) * K + j)],
                    rows_v.at[boff + j],
                    gsem,
                )

        boff = lax.rem(i, 3) * K
        for j in range(K):
            c = i * K + j
            pltpu.make_async_copy(
                table_hbm.at[idx_ref(c)], rows_v.at[boff + j], gsem
            ).wait()
            pltpu.async_copy(
                rows_v.at[boff + j],
                out_hbm.at[pl.ds(base + c * CHUNK, CHUNK)],
                ssem,
            )
        return carry

    lax.fori_loop(0, NGROUP, body, 0)

    for j in range(2 * K):
        pltpu.make_async_copy(
            rows_v.at[0], out_hbm.at[pl.ds(base, CHUNK)], ssem
        ).wait()


def kernel(inputs, embedding):
    idx_flat = _detile_idx(inputs.astype(jnp.int32).T)
    table = jnp.asarray(embedding, jnp.float32)
    out = _embed_gather(idx_flat, table)
    return out.reshape(FIELDS, BATCH, FEATURES).transpose(1, 0, 2)
